# Initial kernel scaffold; baseline (speedup 1.0000x reference)
import jax
import jax.numpy as jnp
from jax.experimental import pallas as pl


def kernel(subdomain_in, subdomain_out, x_idx, y_idx, z_idx, normal_x, normal_y, normal_z):
    raise NotImplementedError("write your pallas kernel here")


# speedup vs baseline: 2.1431x; 2.1431x over previous
import functools

import jax
import jax.numpy as jnp
from jax import lax
from jax.experimental import pallas as pl
from jax.experimental.pallas import tpu as pltpu
from jax.experimental.pallas import tpu_sc as plsc

_N = 128
_DX = 0.05
_WEIGHT = 10.0
_INV = 1.0 / _DX
_NB = 4
_CH_STRIDE = _N * _N * _N
_OFFS = (0, -_N * _N, _N * _N, -_N, _N, -1, 1)
_NOFF = len(_OFFS)


def _make_sc_call(n_pts, nw, p_per_w):
    chunks = p_per_w // 16
    idx_len = _NB * _NOFF * p_per_w
    mesh = plsc.VectorSubcoreMesh(core_axis_name="c", subcore_axis_name="s")

    @functools.partial(
        pl.kernel,
        out_type=jax.ShapeDtypeStruct((nw, 16), jnp.float32),
        mesh=mesh,
        scratch_types=[
            pltpu.VMEM((p_per_w,), jnp.int32),
            pltpu.VMEM((p_per_w,), jnp.int32),
            pltpu.VMEM((p_per_w,), jnp.int32),
            pltpu.VMEM((p_per_w,), jnp.float32),
            pltpu.VMEM((p_per_w,), jnp.float32),
            pltpu.VMEM((p_per_w,), jnp.float32),
            pltpu.VMEM((idx_len,), jnp.int32),
            pltpu.VMEM((idx_len,), jnp.float32),
            pltpu.VMEM((idx_len,), jnp.float32),
            pltpu.VMEM((16,), jnp.float32),
            pltpu.SemaphoreType.DMA,
            pltpu.SemaphoreType.DMA,
        ],
    )
    def sc_call(a_hbm, b_hbm, x_hbm, y_hbm, z_hbm, nx_hbm, ny_hbm, nz_hbm,
                out_hbm, xv, yv, zv, nxv, nyv, nzv, idxv, vin, vout, accv,
                sem_a, sem_b):
        wid = lax.axis_index("s") * 2 + lax.axis_index("c")
        base = wid * p_per_w

        pltpu.sync_copy(x_hbm.at[pl.ds(base, p_per_w)], xv)
        pltpu.sync_copy(y_hbm.at[pl.ds(base, p_per_w)], yv)
        pltpu.sync_copy(z_hbm.at[pl.ds(base, p_per_w)], zv)
        pltpu.sync_copy(nx_hbm.at[pl.ds(base, p_per_w)], nxv)
        pltpu.sync_copy(ny_hbm.at[pl.ds(base, p_per_w)], nyv)
        pltpu.sync_copy(nz_hbm.at[pl.ds(base, p_per_w)], nzv)

        def build(i, carry):
            s = i * 16
            lin = (xv[pl.ds(s, 16)] * (_N * _N)
                   + yv[pl.ds(s, 16)] * _N
                   + zv[pl.ds(s, 16)])
            for c in range(_NB):
                for o in range(_NOFF):
                    idxv[pl.ds((c * _NOFF + o) * p_per_w + s, 16)] = (
                        lin + (c * _CH_STRIDE + _OFFS[o]))
            return carry

        lax.fori_loop(0, chunks, build, 0)

        cp_a = pltpu.async_copy(a_hbm.at[idxv], vin, sem_a)
        cp_b = pltpu.async_copy(b_hbm.at[idxv], vout, sem_b)
        cp_a.wait()
        cp_b.wait()

        lane = lax.iota(jnp.int32, 16)

        def accum(i, acc):
            s = i * 16
            nx = nxv[pl.ds(s, 16)]
            ny = nyv[pl.ds(s, 16)]
            nz = nzv[pl.ds(s, 16)]
            px = nx > 0
            py = ny > 0
            pz = nz > 0
            nzneg = nz < 0
            total = jnp.zeros((16,), jnp.float32)
            for c in range(_NB):
                cb = c * _NOFF * p_per_w
                ci = vin[pl.ds(cb + 0 * p_per_w + s, 16)]
                li = vin[pl.ds(cb + 1 * p_per_w + s, 16)]
                ri = vin[pl.ds(cb + 2 * p_per_w + s, 16)]
                bi = vin[pl.ds(cb + 3 * p_per_w + s, 16)]
                ai = vin[pl.ds(cb + 4 * p_per_w + s, 16)]
                ki = vin[pl.ds(cb + 5 * p_per_w + s, 16)]
                fi = vin[pl.ds(cb + 6 * p_per_w + s, 16)]
                co = vout[pl.ds(cb + 0 * p_per_w + s, 16)]
                lo = vout[pl.ds(cb + 1 * p_per_w + s, 16)]
                ro = vout[pl.ds(cb + 2 * p_per_w + s, 16)]
                bo = vout[pl.ds(cb + 3 * p_per_w + s, 16)]
                ao = vout[pl.ds(cb + 4 * p_per_w + s, 16)]
                ko = vout[pl.ds(cb + 5 * p_per_w + s, 16)]
                fo = vout[pl.ds(cb + 6 * p_per_w + s, 16)]
                d0 = ci - co
                gxi = jnp.where(px, ci - li, ri - ci)
                gxo = jnp.where(px, ro - co, co - lo)
                gyi = jnp.where(py, ci - bi, ai - ci)
                gyo = jnp.where(py, ao - co, co - bo)
                gzi = jnp.where(pz, fi - ci, ci - ki)
                gzo = jnp.where(nzneg, fo - co, co - ko)
                ndi = gxi * nx + gyi * ny + gzi * nz
                ndo = gxo * nx + gyo * ny + gzo * nz
                dn = (ndi - ndo) * _INV
                total = total + d0 * d0 + dn * dn
            g = base + s + lane
            return acc + jnp.where(g < n_pts, total, 0.0)

        acc = lax.fori_loop(0, chunks, accum, jnp.zeros((16,), jnp.float32))
        accv[...] = acc
        pltpu.sync_copy(accv, out_hbm.at[wid])

    return sc_call


def kernel(subdomain_in, subdomain_out, x_idx, y_idx, z_idx,
           normal_x, normal_y, normal_z):
    k = x_idx.shape[0]
    nw = 32
    p_per_w = -(-k // (nw * 16)) * 16
    kpad = nw * p_per_w
    pad = kpad - k

    a = subdomain_in.reshape(-1)
    b = subdomain_out.reshape(-1)
    xp = jnp.pad(x_idx, (0, pad), mode="edge")
    yp = jnp.pad(y_idx, (0, pad), mode="edge")
    zp = jnp.pad(z_idx, (0, pad), mode="edge")
    nxp = jnp.pad(normal_x, (0, pad), mode="edge")
    nyp = jnp.pad(normal_y, (0, pad), mode="edge")
    nzp = jnp.pad(normal_z, (0, pad), mode="edge")

    partials = _make_sc_call(k, nw, p_per_w)(a, b, xp, yp, zp, nxp, nyp, nzp)
    return jnp.sum(partials) * (_WEIGHT / (_NB * k))


# speedup vs baseline: 2.4546x; 1.1453x over previous
import functools

import jax
import jax.numpy as jnp
from jax import lax
from jax.experimental import pallas as pl
from jax.experimental.pallas import tpu as pltpu
from jax.experimental.pallas import tpu_sc as plsc

_N = 128
_DX = 0.05
_WEIGHT = 10.0
_INV = 1.0 / _DX
_NB = 4
_CH_STRIDE = _N * _N * _N
_OFFS = (0, -_N * _N, _N * _N, -_N, _N, -1, 1)
_NOFF = len(_OFFS)


def _make_sc_call(n_pts, nw, p_per_w):
    chunks = p_per_w // 16
    idx_len = _NB * _NOFF * p_per_w
    mesh = plsc.VectorSubcoreMesh(core_axis_name="c", subcore_axis_name="s")

    @functools.partial(
        pl.kernel,
        out_type=jax.ShapeDtypeStruct((nw, 16), jnp.float32),
        mesh=mesh,
        scratch_types=[
            pltpu.VMEM((p_per_w,), jnp.int32),
            pltpu.VMEM((p_per_w,), jnp.int32),
            pltpu.VMEM((p_per_w,), jnp.int32),
            pltpu.VMEM((p_per_w,), jnp.float32),
            pltpu.VMEM((p_per_w,), jnp.float32),
            pltpu.VMEM((p_per_w,), jnp.float32),
            pltpu.VMEM((idx_len,), jnp.int32),
            pltpu.VMEM((idx_len,), jnp.float32),
            pltpu.VMEM((idx_len,), jnp.float32),
            pltpu.VMEM((16,), jnp.float32),
            pltpu.SemaphoreType.DMA,
            pltpu.SemaphoreType.DMA,
        ],
    )
    def sc_call(a_hbm, b_hbm, pts_hbm, nrm_hbm, out_hbm, xv, yv, zv,
                nxv, nyv, nzv, idxv, vin, vout, accv, sem_a, sem_b):
        wid = lax.axis_index("s") * 2 + lax.axis_index("c")
        base = wid * p_per_w

        kpad = nw * p_per_w
        for r, dst in enumerate((xv, yv, zv)):
            pltpu.sync_copy(pts_hbm.at[pl.ds(r * kpad + base, p_per_w)], dst)
        for r, dst in enumerate((nxv, nyv, nzv)):
            pltpu.sync_copy(nrm_hbm.at[pl.ds(r * kpad + base, p_per_w)], dst)

        def build(i, carry):
            s = i * 16
            lin = (xv[pl.ds(s, 16)] * (_N * _N)
                   + yv[pl.ds(s, 16)] * _N
                   + zv[pl.ds(s, 16)])
            for c in range(_NB):
                for o in range(_NOFF):
                    idxv[pl.ds((c * _NOFF + o) * p_per_w + s, 16)] = (
                        lin + (c * _CH_STRIDE + _OFFS[o]))
            return carry

        lax.fori_loop(0, chunks, build, 0)

        cp_a = pltpu.async_copy(a_hbm.at[idxv], vin, sem_a)
        cp_b = pltpu.async_copy(b_hbm.at[idxv], vout, sem_b)
        cp_a.wait()
        cp_b.wait()

        lane = lax.iota(jnp.int32, 16)

        def accum(i, acc):
            s = i * 16
            nx = nxv[pl.ds(s, 16)]
            ny = nyv[pl.ds(s, 16)]
            nz = nzv[pl.ds(s, 16)]
            px = nx > 0
            py = ny > 0
            pz = nz > 0
            nzneg = nz < 0
            total = jnp.zeros((16,), jnp.float32)
            for c in range(_NB):
                cb = c * _NOFF * p_per_w
                ci = vin[pl.ds(cb + 0 * p_per_w + s, 16)]
                li = vin[pl.ds(cb + 1 * p_per_w + s, 16)]
                ri = vin[pl.ds(cb + 2 * p_per_w + s, 16)]
                bi = vin[pl.ds(cb + 3 * p_per_w + s, 16)]
                ai = vin[pl.ds(cb + 4 * p_per_w + s, 16)]
                ki = vin[pl.ds(cb + 5 * p_per_w + s, 16)]
                fi = vin[pl.ds(cb + 6 * p_per_w + s, 16)]
                co = vout[pl.ds(cb + 0 * p_per_w + s, 16)]
                lo = vout[pl.ds(cb + 1 * p_per_w + s, 16)]
                ro = vout[pl.ds(cb + 2 * p_per_w + s, 16)]
                bo = vout[pl.ds(cb + 3 * p_per_w + s, 16)]
                ao = vout[pl.ds(cb + 4 * p_per_w + s, 16)]
                ko = vout[pl.ds(cb + 5 * p_per_w + s, 16)]
                fo = vout[pl.ds(cb + 6 * p_per_w + s, 16)]
                d0 = ci - co
                gxi = jnp.where(px, ci - li, ri - ci)
                gxo = jnp.where(px, ro - co, co - lo)
                gyi = jnp.where(py, ci - bi, ai - ci)
                gyo = jnp.where(py, ao - co, co - bo)
                gzi = jnp.where(pz, fi - ci, ci - ki)
                gzo = jnp.where(nzneg, fo - co, co - ko)
                ndi = gxi * nx + gyi * ny + gzi * nz
                ndo = gxo * nx + gyo * ny + gzo * nz
                dn = (ndi - ndo) * _INV
                total = total + d0 * d0 + dn * dn
            g = base + s + lane
            return acc + jnp.where(g < n_pts, total, 0.0)

        acc = lax.fori_loop(0, chunks, accum, jnp.zeros((16,), jnp.float32))
        accv[...] = acc
        pltpu.sync_copy(accv, out_hbm.at[wid])

    return sc_call


def kernel(subdomain_in, subdomain_out, x_idx, y_idx, z_idx,
           normal_x, normal_y, normal_z):
    k = x_idx.shape[0]
    nw = 32
    p_per_w = -(-k // (nw * 16)) * 16
    kpad = nw * p_per_w
    pad = kpad - k

    a = subdomain_in.reshape(-1)
    b = subdomain_out.reshape(-1)
    pts = jnp.pad(jnp.stack([x_idx, y_idx, z_idx]), ((0, 0), (0, pad)),
                  constant_values=64).reshape(-1)
    nrm = jnp.pad(jnp.stack([normal_x, normal_y, normal_z]),
                  ((0, 0), (0, pad))).reshape(-1)

    partials = _make_sc_call(k, nw, p_per_w)(a, b, pts, nrm)
    return jnp.sum(partials) * (_WEIGHT / (_NB * k))
